# baseline (speedup 1.0000x reference)
import functools

import jax
import jax.numpy as jnp
from jax import lax
from jax.experimental import pallas as pl
from jax.experimental.pallas import tpu as pltpu
from jax.experimental.pallas import tpu_sc as plsc

BATCH = 4096
HIST = 200
D = 64
DP = 128
NC, NS = 2, 16
NW = NC * NS
RPW = BATCH // NW
NBUF = 3
L = 16

_MESH = plsc.VectorSubcoreMesh(core_axis_name="c", subcore_axis_name="s")


@functools.partial(
    pl.kernel,
    mesh=_MESH,
    out_type=jax.ShapeDtypeStruct((HIST, D, BATCH), jnp.float32),
    scratch_types=[
        pltpu.VMEM((RPW * HIST,), jnp.int32),
        pltpu.VMEM((RPW * HIST,), jnp.int32),
        pltpu.VMEM((NBUF, RPW, DP), jnp.float32),
        pltpu.VMEM((NBUF, D, RPW), jnp.float32),
        pltpu.SemaphoreType.DMA((NBUF,)),
        pltpu.SemaphoreType.DMA((NBUF,)),
    ],
    compiler_params=pltpu.CompilerParams(
        use_tc_tiling_on_sc=True, needs_layout_passes=False),
)
def _gather_kernel(idx_hbm, table_hbm, out_hbm, idx_v, idx_t, rows_v,
                   trans_v, sg, so):
    wid = lax.axis_index("s") * NC + lax.axis_index("c")
    b0 = wid * RPW
    lanes = lax.iota(jnp.int32, L)

    pltpu.sync_copy(idx_hbm.at[pl.ds(b0 * HIST, RPW * HIST)], idx_v)

    def tr_idx(t, carry):
        for k in range(RPW // L):
            pos = lanes * HIST + (k * L * HIST + t)
            vals = plsc.load_gather(idx_v, [pos])
            idx_t[pl.ds(t * RPW + k * L, L)] = vals
        return carry

    lax.fori_loop(0, HIST, tr_idx, 0)

    def gather(t, b):
        return pltpu.make_async_copy(
            table_hbm.at[idx_t.at[pl.ds(t * RPW, RPW)]], rows_v.at[b],
            sg.at[b])

    def store(t, b):
        return pltpu.make_async_copy(
            trans_v.at[b], out_hbm.at[t, :, pl.ds(b0, RPW)], so.at[b])

    def transpose(b):
        rows = rows_v.at[b]
        trans = trans_v.at[b]

        def tr_d(d, carry):
            for k in range(RPW // L):
                vals = plsc.load_gather(
                    rows, [k * L + lanes, jnp.full((L,), d, jnp.int32)])
                trans[d, pl.ds(k * L, L)] = vals
            return carry

        lax.fori_loop(0, D, tr_d, 0)

    gather(0, 0).start()
    gather(1, 1).start()
    for t in range(3):
        gather(t, t).wait()
        gather(t + 2, (t + 2) % NBUF).start()
        transpose(t)
        store(t, t).start()

    def step(ts, carry):
        for j in range(NBUF):
            t = ts * NBUF + j
            gather(t, j).wait()
            gather(t + 2, (j + 2) % NBUF).start()
            store(0, j).wait()
            transpose(j)
            store(t, j).start()
        return carry

    lax.fori_loop(1, (HIST - 2) // NBUF, step, 0)

    for t in (HIST - 2, HIST - 1):
        j = t % NBUF
        gather(t, j).wait()
        store(0, j).wait()
        transpose(j)
        store(t, j).start()
    for t in (HIST - 3, HIST - 2, HIST - 1):
        store(0, t % NBUF).wait()


def kernel(indices, table):
    idx_flat = indices.reshape(-1)
    table_p = jnp.pad(table, ((0, 0), (0, DP - D)))
    out_t = _gather_kernel(idx_flat, table_p)
    return out_t.transpose(2, 0, 1)


# speedup vs baseline: 1.7529x; 1.7529x over previous
import functools

import jax
import jax.numpy as jnp
from jax import lax
from jax.experimental import pallas as pl
from jax.experimental.pallas import tpu as pltpu
from jax.experimental.pallas import tpu_sc as plsc

BATCH = 4096
HIST = 200
D = 64
DP = 128
V = 1000000
NC, NS = 2, 16
NW = NC * NS
RPW = BATCH // NW
NBUF = 4
WR = 31248
WBLK = 248
WN = WR // WBLK
WTAIL = V - NW * WR

_MESH = plsc.VectorSubcoreMesh(core_axis_name="c", subcore_axis_name="s")


def _run_pipeline(n, mk_a, mk_b):
    def stat(i):
        j = i % NBUF
        mk_a(i, j).wait()
        if i >= 2:
            mk_b(0, (j + 2) % NBUF).wait()
        if i + 2 < n:
            mk_a(i + 2, (i + 2) % NBUF).start()
        mk_b(i, j).start()

    mk_a(0, 0).start()
    mk_a(1, 1).start()
    for i in range(NBUF):
        stat(i)

    t_hi = ((n - 2) // NBUF) * NBUF

    def step(ts, carry):
        for j in range(NBUF):
            i = ts * NBUF + j
            mk_a(0, j).wait()
            mk_b(0, (j + 2) % NBUF).wait()
            mk_a(i + 2, (i + 2) % NBUF).start()
            mk_b(i, j).start()
        return carry

    lax.fori_loop(1, t_hi // NBUF, step, 0)
    for i in range(t_hi, n):
        stat(i)
    mk_b(0, (n - 2) % NBUF).wait()
    mk_b(0, (n - 1) % NBUF).wait()


@functools.partial(
    pl.kernel,
    mesh=_MESH,
    out_type=jax.ShapeDtypeStruct((V, DP), jnp.float32),
    scratch_types=[
        pltpu.VMEM((NBUF, WBLK, D), jnp.float32),
        pltpu.VMEM((WTAIL, D), jnp.float32),
        pltpu.SemaphoreType.DMA((NBUF,)),
        pltpu.SemaphoreType.DMA((NBUF,)),
    ],
    compiler_params=pltpu.CompilerParams(
        use_tc_tiling_on_sc=True, needs_layout_passes=False),
)
def _widen_kernel(tab_hbm, out_hbm, buf, tbuf, si, so):
    wid = lax.axis_index("s") * NC + lax.axis_index("c")
    r0 = wid * WR

    def load(g, b):
        return pltpu.make_async_copy(
            tab_hbm.at[pl.ds(r0 + g * WBLK, WBLK), :], buf.at[b], si.at[b])

    def store(g, b):
        return pltpu.make_async_copy(
            buf.at[b], out_hbm.at[pl.ds(r0 + g * WBLK, WBLK), pl.ds(0, D)],
            so.at[b])

    _run_pipeline(WN, load, store)

    @pl.when(wid == 0)
    def _tail():
        pltpu.sync_copy(tab_hbm.at[pl.ds(NW * WR, WTAIL), :], tbuf)
        pltpu.sync_copy(tbuf, out_hbm.at[pl.ds(NW * WR, WTAIL), pl.ds(0, D)])


@functools.partial(
    pl.kernel,
    mesh=_MESH,
    out_type=jax.ShapeDtypeStruct((BATCH, HIST, DP), jnp.float32),
    scratch_types=[
        pltpu.VMEM((RPW * HIST,), jnp.int32),
        pltpu.VMEM((NBUF, HIST, DP), jnp.float32),
        pltpu.SemaphoreType.DMA((NBUF,)),
        pltpu.SemaphoreType.DMA((NBUF,)),
    ],
    compiler_params=pltpu.CompilerParams(
        use_tc_tiling_on_sc=True, needs_layout_passes=False),
)
def _gather_kernel(idx_hbm, table_hbm, out_hbm, idx_v, rows_v, sg, so):
    wid = lax.axis_index("s") * NC + lax.axis_index("c")
    b0 = wid * RPW

    pltpu.sync_copy(idx_hbm.at[pl.ds(b0 * HIST, RPW * HIST)], idx_v)

    def gather(i, b):
        return pltpu.make_async_copy(
            table_hbm.at[idx_v.at[pl.ds(i * HIST, HIST)]], rows_v.at[b],
            sg.at[b])

    def store(i, b):
        return pltpu.make_async_copy(
            rows_v.at[b], out_hbm.at[b0 + i], so.at[b])

    _run_pipeline(RPW, gather, store)


def kernel(indices, table):
    idx_flat = indices.reshape(-1)
    table_p = jnp.pad(table, ((0, 0), (0, DP - D)))
    out_p = _gather_kernel(idx_flat, table_p)
    return out_p[..., :D]
